# baseline (speedup 1.0000x reference)
import functools

import jax
import jax.numpy as jnp
from jax import lax
from jax.experimental import pallas as pl
from jax.experimental.pallas import tpu as pltpu
from jax.experimental.pallas import tpu_sc as plsc

F32 = jnp.float32
K = 1024
D = 64


def _enc1_body(x1_ref, w_ref, b_ref, out_ref):
    a = x1_ref[0].reshape(128 * 128, 16)
    y = jnp.dot(a, w_ref[...], preferred_element_type=F32) + b_ref[0:1, :]
    out_ref[0] = jnp.maximum(y, 0.0).reshape(128, 128, 16)


def _enc23_body(v_ref, w2_ref, b2_ref, w3_ref, b3_ref, out_ref):
    acc = jnp.zeros((64 * 64, 32), F32)
    for dy in range(2):
        for dx in range(2):
            patch = v_ref[0, dy:dy + 64, dx:dx + 64, :].reshape(64 * 64, 64)
            w = w2_ref[pl.ds(64 * (2 * dy + dx), 64), :]
            acc = acc + jnp.dot(patch, w, preferred_element_type=F32)
    h = jnp.maximum(acc + b2_ref[0:1, :], 0.0)
    z = jnp.dot(h, w3_ref[...], preferred_element_type=F32) + b3_ref[0:1, :]
    out_ref[...] = z


def _vq_body(z_ref, ct_ref, idx_ref, loss_ref):
    i = pl.program_id(0)
    zb = z_ref[...]
    ct = ct_ref[...]
    scores = jnp.dot(zb, ct, preferred_element_type=F32)
    zsq = jnp.sum(zb * zb, axis=1, keepdims=True)
    csq = jnp.sum(ct * ct, axis=0, keepdims=True)
    dist = (zsq + csq) - 2.0 * scores
    m = jnp.min(dist, axis=1, keepdims=True)
    lanes = lax.broadcasted_iota(jnp.int32, dist.shape, 1)
    idx = jnp.min(jnp.where(dist == m, lanes, jnp.int32(K)), axis=1,
                  keepdims=True)
    idx_ref[...] = idx
    part = jnp.sum(m)

    @pl.when(i == 0)
    def _init():
        loss_ref[0, 0] = part

    @pl.when(i > 0)
    def _acc():
        loss_ref[0, 0] = loss_ref[0, 0] + part


def _dec12_body(q_ref, w4_ref, b4_ref, w5_ref, b5_ref, out_ref, h_scr):
    acc = jnp.zeros((64 * 64, 32), F32)
    for sy in range(3):
        for sx in range(3):
            patch = q_ref[0, sy:sy + 64, sx:sx + 64, :].reshape(64 * 64, 64)
            w = w4_ref[pl.ds(64 * (3 * sy + sx), 64), :]
            acc = acc + jnp.dot(patch, w, preferred_element_type=F32)
    h = jnp.maximum(acc + b4_ref[0:1, :], 0.0)
    h_scr[...] = jnp.zeros((66, 66, 32), F32)
    h_scr[1:65, 1:65, :] = h.reshape(64, 64, 32)
    acc2 = jnp.zeros((64 * 64, 64), F32)
    for sy in range(3):
        for sx in range(3):
            patch = h_scr[sy:sy + 64, sx:sx + 64, :].reshape(64 * 64, 32)
            w = w5_ref[pl.ds(32 * (3 * sy + sx), 32), :]
            acc2 = acc2 + jnp.dot(patch, w, preferred_element_type=F32)
    y = jnp.maximum(acc2 + b5_ref[0:1, :], 0.0)
    out_ref[0] = y.reshape(64, 64, 64)


def _dec3_body(h_ref, w_ref, b_ref, out_ref):
    for j in range(2):
        acc = jnp.zeros((128, 256), F32)
        for sy in range(3):
            a = h_ref[0, sy:sy + 128, pl.ds(1024 * j, 1056)]
            acc = acc + jnp.dot(a, w_ref[pl.ds(1056 * sy, 1056), :],
                                preferred_element_type=F32)
        yv = 1.0 / (1.0 + jnp.exp(-(acc + b_ref[0:1, :])))
        out_ref[0, :, :, j, :] = yv.reshape(128, 2, 128)


_NC, _NS = 2, 16
_NW = _NC * _NS
_BPW = 32768 // _NW
_HALF = _BPW // 2


def _sc_gather(codebook, idx2d):
    mesh = plsc.VectorSubcoreMesh(core_axis_name="c", subcore_axis_name="s")

    @functools.partial(
        pl.kernel, mesh=mesh,
        out_type=jax.ShapeDtypeStruct((32768 * D,), F32),
        scratch_types=[
            pltpu.VMEM((_BPW,), jnp.int32),
            pltpu.VMEM((K * D,), F32),
            pltpu.VMEM((_HALF * D,), F32),
        ],
        compiler_params=pltpu.CompilerParams(needs_layout_passes=False),
    )
    def gather(table_hbm, idx_hbm, out_hbm, idx_v, tab_v, out_v):
        wid = lax.axis_index("s") * _NC + lax.axis_index("c")
        base = wid * _BPW
        pltpu.sync_copy(table_hbm, tab_v)
        pltpu.sync_copy(idx_hbm.at[wid], idx_v)
        lanes16 = lax.broadcasted_iota(jnp.int32, (16,), 0)
        for h in range(2):
            def body(g, carry):
                iv64 = idx_v[pl.ds(h * _HALF + g * 16, 16)] * D
                rrel64 = (g * 16 + lanes16) * D
                for c in range(D):
                    vals = plsc.load_gather(tab_v, [iv64 + c])
                    plsc.store_scatter(out_v, [rrel64 + c], vals)
                return carry
            lax.fori_loop(0, _HALF // 16, body, 0)
            pltpu.sync_copy(out_v, out_hbm.at[pl.ds((base + h * _HALF) * D,
                                                    _HALF * D)])

    return gather(codebook.reshape(K * D), idx2d)


_TAPS = {(0, 0): 3, (0, 1): 1, (1, 1): 2, (1, 2): 0}


def _s2d(a):
    n, h2, w2, c = a.shape
    a = a.reshape(n, h2 // 2, 2, w2 // 2, 2, c)
    return jnp.transpose(a, (0, 1, 3, 2, 4, 5)).reshape(
        n, h2 // 2, w2 // 2, 4 * c)


def _enc_stride2_weights(w):
    cout, cin = w.shape[0], w.shape[1]
    a = jnp.transpose(w, (2, 3, 1, 0))
    a = a.reshape(2, 2, 2, 2, cin, cout)
    a = jnp.transpose(a, (0, 2, 1, 3, 4, 5))
    return a.reshape(16 * cin, cout)


def _dect_weights(w):
    cin, cout = w.shape[0], w.shape[1]
    full = jnp.zeros((3, 3, cin, 2, 2, cout), F32)
    for (py, sy), ky in _TAPS.items():
        for (px, sx), kx in _TAPS.items():
            full = full.at[sy, sx, :, py, px, :].set(w[:, :, ky, kx])
    return full.reshape(9 * cin, 4 * cout)


def _toeplitz_weights(w):
    cin = w.shape[0]
    wsel = jnp.zeros((cin, 2, 2, 3, 3), F32)
    for (qy, sy), ky in _TAPS.items():
        for (qx, sx), kx in _TAPS.items():
            wsel = wsel.at[:, qy, qx, sy, sx].set(w[:, 0, ky, kx])
    ind = (lax.broadcasted_iota(jnp.int32, (3, 66, 64), 1) ==
           lax.broadcasted_iota(jnp.int32, (3, 66, 64), 2) +
           lax.broadcasted_iota(jnp.int32, (3, 66, 64), 0)).astype(F32)
    wt = jnp.einsum('xpr,cyqsx->spcyrq', ind, wsel)
    return wt.reshape(3 * 66 * cin, 256)


def kernel(x, enc_w1, enc_b1, enc_w2, enc_b2, enc_w3, enc_b3, codebook,
           dec_w1, dec_b1, dec_w2, dec_b2, dec_w3, dec_b3):
    n = x.shape[0]
    xp = jnp.pad(x[:, 0], ((0, 0), (1, 1), (1, 1)))
    v1 = _s2d(xp[..., None])
    x1 = jnp.concatenate(
        [v1[:, dy:dy + 128, dx:dx + 128, :]
         for dy in range(2) for dx in range(2)], axis=-1)
    w1 = _enc_stride2_weights(enc_w1)
    y1 = pl.pallas_call(
        _enc1_body,
        grid=(n,),
        in_specs=[
            pl.BlockSpec((1, 128, 128, 16), lambda i: (i, 0, 0, 0)),
            pl.BlockSpec((16, 16), lambda i: (0, 0)),
            pl.BlockSpec((1, 16), lambda i: (0, 0)),
        ],
        out_specs=pl.BlockSpec((1, 128, 128, 16), lambda i: (i, 0, 0, 0)),
        out_shape=jax.ShapeDtypeStruct((n, 128, 128, 16), F32),
    )(x1, w1, enc_b1[None, :])

    v2 = _s2d(jnp.pad(y1, ((0, 0), (1, 1), (1, 1), (0, 0))))
    w2 = _enc_stride2_weights(enc_w2)
    w3 = jnp.transpose(enc_w3[:, :, 0, 0])
    z = pl.pallas_call(
        _enc23_body,
        grid=(n,),
        in_specs=[
            pl.BlockSpec((1, 65, 65, 64), lambda i: (i, 0, 0, 0)),
            pl.BlockSpec((256, 32), lambda i: (0, 0)),
            pl.BlockSpec((1, 32), lambda i: (0, 0)),
            pl.BlockSpec((32, 64), lambda i: (0, 0)),
            pl.BlockSpec((1, 64), lambda i: (0, 0)),
        ],
        out_specs=pl.BlockSpec((64 * 64, 64), lambda i: (i, 0)),
        out_shape=jax.ShapeDtypeStruct((n * 64 * 64, D), F32),
    )(v2, w2, enc_b2[None, :], w3, enc_b3[None, :])

    bm = 512
    nblk = (n * 64 * 64) // bm
    ct = jnp.transpose(codebook)
    idx2d, loss_sum = pl.pallas_call(
        _vq_body,
        grid=(nblk,),
        in_specs=[
            pl.BlockSpec((bm, D), lambda i: (i, 0)),
            pl.BlockSpec((D, K), lambda i: (0, 0)),
        ],
        out_specs=[
            pl.BlockSpec((bm, 1), lambda i: (i, 0)),
            pl.BlockSpec((1, 1), lambda i: (0, 0),
                         memory_space=pltpu.SMEM),
        ],
        out_shape=[
            jax.ShapeDtypeStruct((n * 64 * 64, 1), jnp.int32),
            jax.ShapeDtypeStruct((1, 1), F32),
        ],
    )(z, ct)
    nel = jnp.float32(n * 64 * 64 * D)
    loss = (loss_sum[0, 0] / nel) * jnp.float32(1.25)
    indices = idx2d.reshape(n, 64, 64)

    q = _sc_gather(codebook, idx2d.reshape(_NW, _BPW))
    q = q.reshape(32768, D)

    qp = jnp.pad(q.reshape(n, 64, 64, D),
                 ((0, 0), (1, 1), (1, 1), (0, 0)))
    w4 = jnp.transpose(dec_w1, (2, 3, 1, 0)).reshape(9 * 64, 32)
    w5 = _dect_weights(dec_w2)
    b5 = jnp.tile(dec_b2, 4)[None, :]
    y5 = pl.pallas_call(
        _dec12_body,
        grid=(n,),
        in_specs=[
            pl.BlockSpec((1, 66, 66, 64), lambda i: (i, 0, 0, 0)),
            pl.BlockSpec((9 * 64, 32), lambda i: (0, 0)),
            pl.BlockSpec((1, 32), lambda i: (0, 0)),
            pl.BlockSpec((9 * 32, 64), lambda i: (0, 0)),
            pl.BlockSpec((1, 64), lambda i: (0, 0)),
        ],
        out_specs=pl.BlockSpec((1, 64, 64, 64), lambda i: (i, 0, 0, 0)),
        out_shape=jax.ShapeDtypeStruct((n, 64, 64, 64), F32),
        scratch_shapes=[pltpu.VMEM((66, 66, 32), F32)],
    )(qp, w4, dec_b1[None, :], w5, b5)

    h2f = jnp.pad(
        jnp.transpose(y5.reshape(n, 64, 64, 2, 2, 16),
                      (0, 1, 3, 2, 4, 5)).reshape(n, 128, 2048),
        ((0, 0), (1, 1), (16, 16)))
    w6 = _toeplitz_weights(dec_w3)
    b6 = jnp.broadcast_to(dec_b3[0], (1, 256)).astype(F32)
    y6 = pl.pallas_call(
        _dec3_body,
        grid=(n,),
        in_specs=[
            pl.BlockSpec((1, 130, 2080), lambda i: (i, 0, 0)),
            pl.BlockSpec((3168, 256), lambda i: (0, 0)),
            pl.BlockSpec((1, 256), lambda i: (0, 0)),
        ],
        out_specs=pl.BlockSpec((1, 128, 2, 2, 128), lambda i: (i, 0, 0, 0, 0)),
        out_shape=jax.ShapeDtypeStruct((n, 128, 2, 2, 128), F32),
    )(h2f, w6, b6)
    recon = y6.reshape(n, 1, 256, 256)

    return (recon, loss, indices)


# speedup vs baseline: 1.0907x; 1.0907x over previous
import functools

import jax
import jax.numpy as jnp
from jax import lax
from jax.experimental import pallas as pl
from jax.experimental.pallas import tpu as pltpu
from jax.experimental.pallas import tpu_sc as plsc

F32 = jnp.float32
K = 1024
D = 64


def _enc1_body(x1_ref, w_ref, b_ref, out_ref):
    a = x1_ref[0].reshape(128 * 128, 16)
    y = jnp.dot(a, w_ref[...], preferred_element_type=F32) + b_ref[0:1, :]
    out_ref[0] = jnp.maximum(y, 0.0).reshape(128, 128, 16)


def _enc23_body(v_ref, w2_ref, b2_ref, w3_ref, b3_ref, out_ref):
    acc = jnp.zeros((64 * 64, 32), F32)
    for dy in range(2):
        for dx in range(2):
            patch = v_ref[0, dy:dy + 64, dx:dx + 64, :].reshape(64 * 64, 64)
            w = w2_ref[pl.ds(64 * (2 * dy + dx), 64), :]
            acc = acc + jnp.dot(patch, w, preferred_element_type=F32)
    h = jnp.maximum(acc + b2_ref[0:1, :], 0.0)
    z = jnp.dot(h, w3_ref[...], preferred_element_type=F32) + b3_ref[0:1, :]
    out_ref[...] = z


def _vq_body(z_ref, ct_ref, idx_ref, loss_ref):
    i = pl.program_id(0)
    zb = z_ref[...]
    ct = ct_ref[...]
    scores = jnp.dot(zb, ct, preferred_element_type=F32)
    zsq = jnp.sum(zb * zb, axis=1, keepdims=True)
    csq = jnp.sum(ct * ct, axis=0, keepdims=True)
    dist = (zsq + csq) - 2.0 * scores
    m = jnp.min(dist, axis=1, keepdims=True)
    lanes = lax.broadcasted_iota(jnp.int32, dist.shape, 1)
    idx = jnp.min(jnp.where(dist == m, lanes, jnp.int32(K)), axis=1,
                  keepdims=True)
    idx_ref[...] = idx
    part = jnp.sum(m)

    @pl.when(i == 0)
    def _init():
        loss_ref[0, 0] = part

    @pl.when(i > 0)
    def _acc():
        loss_ref[0, 0] = loss_ref[0, 0] + part


def _dec12_body(q_ref, w4_ref, b4_ref, w5_ref, b5_ref, out_ref, h_scr):
    acc = jnp.zeros((64 * 64, 32), F32)
    for sy in range(3):
        for sx in range(3):
            patch = q_ref[0, sy:sy + 64, sx:sx + 64, :].reshape(64 * 64, 64)
            w = w4_ref[pl.ds(64 * (3 * sy + sx), 64), :]
            acc = acc + jnp.dot(patch, w, preferred_element_type=F32)
    h = jnp.maximum(acc + b4_ref[0:1, :], 0.0)
    h_scr[...] = jnp.zeros((66, 66, 32), F32)
    h_scr[1:65, 1:65, :] = h.reshape(64, 64, 32)
    acc2 = jnp.zeros((64 * 64, 64), F32)
    for sy in range(3):
        for sx in range(3):
            patch = h_scr[sy:sy + 64, sx:sx + 64, :].reshape(64 * 64, 32)
            w = w5_ref[pl.ds(32 * (3 * sy + sx), 32), :]
            acc2 = acc2 + jnp.dot(patch, w, preferred_element_type=F32)
    y = jnp.maximum(acc2 + b5_ref[0:1, :], 0.0).reshape(64, 64, 2, 2, 16)
    out_ref[0] = jnp.zeros((65, 2, 2080), F32)
    out_ref[0, 0:64, 1, 16:2064] = y[:, :, 0].reshape(64, 2048)
    out_ref[0, 1:65, 0, 16:2064] = y[:, :, 1].reshape(64, 2048)


def _dec3_body(h_ref, w_ref, b_ref, out_ref):
    for j in range(2):
        acc = jnp.zeros((128, 256), F32)
        for sy in range(3):
            a = h_ref[0, sy:sy + 128, pl.ds(1024 * j, 1056)]
            acc = acc + jnp.dot(a, w_ref[pl.ds(1056 * sy, 1056), :],
                                preferred_element_type=F32)
        yv = 1.0 / (1.0 + jnp.exp(-(acc + b_ref[0:1, :])))
        out_ref[0, :, :, j, :] = yv.reshape(128, 2, 128)


_NC, _NS = 2, 16
_NW = _NC * _NS
_BPW = 32768 // _NW
_HALF = _BPW // 2


def _sc_gather(codebook, idx2d):
    mesh = plsc.VectorSubcoreMesh(core_axis_name="c", subcore_axis_name="s")

    @functools.partial(
        pl.kernel, mesh=mesh,
        out_type=jax.ShapeDtypeStruct((32768 * D,), F32),
        scratch_types=[
            pltpu.VMEM((_BPW,), jnp.int32),
            pltpu.VMEM((K * D,), F32),
            pltpu.VMEM((_HALF * D,), F32),
        ],
        compiler_params=pltpu.CompilerParams(needs_layout_passes=False),
    )
    def gather(table_hbm, idx_hbm, out_hbm, idx_v, tab_v, out_v):
        wid = lax.axis_index("s") * _NC + lax.axis_index("c")
        base = wid * _BPW
        pltpu.sync_copy(table_hbm, tab_v)
        pltpu.sync_copy(idx_hbm.at[wid], idx_v)
        lanes16 = lax.broadcasted_iota(jnp.int32, (16,), 0)
        for h in range(2):
            def body(g, carry):
                iv64 = idx_v[pl.ds(h * _HALF + g * 16, 16)] * D
                rrel64 = (g * 16 + lanes16) * D
                for c in range(D):
                    vals = plsc.load_gather(tab_v, [iv64 + c])
                    plsc.store_scatter(out_v, [rrel64 + c], vals)
                return carry
            lax.fori_loop(0, _HALF // 16, body, 0)
            pltpu.sync_copy(out_v, out_hbm.at[pl.ds((base + h * _HALF) * D,
                                                    _HALF * D)])

    return gather(codebook.reshape(K * D), idx2d)


_TAPS = {(0, 0): 3, (0, 1): 1, (1, 1): 2, (1, 2): 0}


def _s2d(a):
    n, h2, w2, c = a.shape
    a = a.reshape(n, h2 // 2, 2, w2 // 2, 2, c)
    return jnp.transpose(a, (0, 1, 3, 2, 4, 5)).reshape(
        n, h2 // 2, w2 // 2, 4 * c)


def _enc_stride2_weights(w):
    cout, cin = w.shape[0], w.shape[1]
    a = jnp.transpose(w, (2, 3, 1, 0))
    a = a.reshape(2, 2, 2, 2, cin, cout)
    a = jnp.transpose(a, (0, 2, 1, 3, 4, 5))
    return a.reshape(16 * cin, cout)


def _dect_weights(w):
    cin, cout = w.shape[0], w.shape[1]
    full = jnp.zeros((3, 3, cin, 2, 2, cout), F32)
    for (py, sy), ky in _TAPS.items():
        for (px, sx), kx in _TAPS.items():
            full = full.at[sy, sx, :, py, px, :].set(w[:, :, ky, kx])
    return full.reshape(9 * cin, 4 * cout)


def _toeplitz_weights(w):
    cin = w.shape[0]
    wsel = jnp.zeros((cin, 2, 2, 3, 3), F32)
    for (qy, sy), ky in _TAPS.items():
        for (qx, sx), kx in _TAPS.items():
            wsel = wsel.at[:, qy, qx, sy, sx].set(w[:, 0, ky, kx])
    ind = (lax.broadcasted_iota(jnp.int32, (3, 66, 64), 1) ==
           lax.broadcasted_iota(jnp.int32, (3, 66, 64), 2) +
           lax.broadcasted_iota(jnp.int32, (3, 66, 64), 0)).astype(F32)
    wt = jnp.einsum('xpr,cyqsx->spcyrq', ind, wsel)
    return wt.reshape(3 * 66 * cin, 256)


def kernel(x, enc_w1, enc_b1, enc_w2, enc_b2, enc_w3, enc_b3, codebook,
           dec_w1, dec_b1, dec_w2, dec_b2, dec_w3, dec_b3):
    n = x.shape[0]
    xp = jnp.pad(x[:, 0], ((0, 0), (1, 1), (1, 1)))
    v1 = _s2d(xp[..., None])
    x1 = jnp.concatenate(
        [v1[:, dy:dy + 128, dx:dx + 128, :]
         for dy in range(2) for dx in range(2)], axis=-1)
    w1 = _enc_stride2_weights(enc_w1)
    y1 = pl.pallas_call(
        _enc1_body,
        grid=(n,),
        in_specs=[
            pl.BlockSpec((1, 128, 128, 16), lambda i: (i, 0, 0, 0)),
            pl.BlockSpec((16, 16), lambda i: (0, 0)),
            pl.BlockSpec((1, 16), lambda i: (0, 0)),
        ],
        out_specs=pl.BlockSpec((1, 128, 128, 16), lambda i: (i, 0, 0, 0)),
        out_shape=jax.ShapeDtypeStruct((n, 128, 128, 16), F32),
    )(x1, w1, enc_b1[None, :])

    v2 = _s2d(jnp.pad(y1, ((0, 0), (1, 1), (1, 1), (0, 0))))
    w2 = _enc_stride2_weights(enc_w2)
    w3 = jnp.transpose(enc_w3[:, :, 0, 0])
    z = pl.pallas_call(
        _enc23_body,
        grid=(n,),
        in_specs=[
            pl.BlockSpec((1, 65, 65, 64), lambda i: (i, 0, 0, 0)),
            pl.BlockSpec((256, 32), lambda i: (0, 0)),
            pl.BlockSpec((1, 32), lambda i: (0, 0)),
            pl.BlockSpec((32, 64), lambda i: (0, 0)),
            pl.BlockSpec((1, 64), lambda i: (0, 0)),
        ],
        out_specs=pl.BlockSpec((64 * 64, 64), lambda i: (i, 0)),
        out_shape=jax.ShapeDtypeStruct((n * 64 * 64, D), F32),
    )(v2, w2, enc_b2[None, :], w3, enc_b3[None, :])

    bm = 1024
    nblk = (n * 64 * 64) // bm
    ct = jnp.transpose(codebook)
    idx2d, loss_sum = pl.pallas_call(
        _vq_body,
        grid=(nblk,),
        in_specs=[
            pl.BlockSpec((bm, D), lambda i: (i, 0)),
            pl.BlockSpec((D, K), lambda i: (0, 0)),
        ],
        out_specs=[
            pl.BlockSpec((bm, 1), lambda i: (i, 0)),
            pl.BlockSpec((1, 1), lambda i: (0, 0),
                         memory_space=pltpu.SMEM),
        ],
        out_shape=[
            jax.ShapeDtypeStruct((n * 64 * 64, 1), jnp.int32),
            jax.ShapeDtypeStruct((1, 1), F32),
        ],
    )(z, ct)
    nel = jnp.float32(n * 64 * 64 * D)
    loss = (loss_sum[0, 0] / nel) * jnp.float32(1.25)
    indices = idx2d.reshape(n, 64, 64)

    q = _sc_gather(codebook, idx2d.reshape(_NW, _BPW))
    q = q.reshape(32768, D)

    qp = jnp.pad(q.reshape(n, 64, 64, D),
                 ((0, 0), (1, 1), (1, 1), (0, 0)))
    w4 = jnp.transpose(dec_w1, (2, 3, 1, 0)).reshape(9 * 64, 32)
    w5 = _dect_weights(dec_w2)
    b5 = jnp.tile(dec_b2, 4)[None, :]
    y5 = pl.pallas_call(
        _dec12_body,
        grid=(n,),
        in_specs=[
            pl.BlockSpec((1, 66, 66, 64), lambda i: (i, 0, 0, 0)),
            pl.BlockSpec((9 * 64, 32), lambda i: (0, 0)),
            pl.BlockSpec((1, 32), lambda i: (0, 0)),
            pl.BlockSpec((9 * 32, 64), lambda i: (0, 0)),
            pl.BlockSpec((1, 64), lambda i: (0, 0)),
        ],
        out_specs=pl.BlockSpec((1, 65, 2, 2080), lambda i: (i, 0, 0, 0)),
        out_shape=jax.ShapeDtypeStruct((n, 65, 2, 2080), F32),
        scratch_shapes=[pltpu.VMEM((66, 66, 32), F32)],
    )(qp, w4, dec_b1[None, :], w5, b5)

    h2f = y5.reshape(n, 130, 2080)
    w6 = _toeplitz_weights(dec_w3)
    b6 = jnp.broadcast_to(dec_b3[0], (1, 256)).astype(F32)
    y6 = pl.pallas_call(
        _dec3_body,
        grid=(n,),
        in_specs=[
            pl.BlockSpec((1, 130, 2080), lambda i: (i, 0, 0)),
            pl.BlockSpec((3168, 256), lambda i: (0, 0)),
            pl.BlockSpec((1, 256), lambda i: (0, 0)),
        ],
        out_specs=pl.BlockSpec((1, 128, 2, 2, 128), lambda i: (i, 0, 0, 0, 0)),
        out_shape=jax.ShapeDtypeStruct((n, 128, 2, 2, 128), F32),
    )(h2f, w6, b6)
    recon = y6.reshape(n, 1, 256, 256)

    return (recon, loss, indices)
